# baseline (speedup 1.0000x reference)
import functools

import jax
import jax.numpy as jnp
from jax.experimental import pallas as pl
from jax.experimental.pallas import tpu as pltpu

DIM = 64
CB = 1024
NQ = 8


def _rvq_kernel(z_ref, cb_ref, cn_ref, q_ref, tok_ref, loss_ref):
    resid = z_ref[...]
    blk = resid.shape[0]
    qtot = jnp.zeros_like(resid)
    iota = jax.lax.broadcasted_iota(jnp.int32, (blk, CB), 1)
    idx_cols = []
    loss_parts = []
    for s in range(NQ):
        cb = cb_ref[s]
        rn = jnp.sum(resid * resid, axis=1, keepdims=True)
        mm = jax.lax.dot_general(
            resid, cb, (((1,), (1,)), ((), ())),
            preferred_element_type=jnp.float32)
        dist = (rn - 2.0 * mm) + cn_ref[s][None, :]
        m = jnp.min(dist, axis=1, keepdims=True)
        idx = jnp.min(jnp.where(dist == m, iota, CB), axis=1, keepdims=True)
        oh = (iota == idx).astype(jnp.float32)
        codes = jax.lax.dot_general(
            oh, cb, (((1,), (0,)), ((), ())),
            precision=jax.lax.Precision.HIGHEST,
            preferred_element_type=jnp.float32)
        d = resid - codes
        loss_parts.append(jnp.sum(d * d))
        qtot = qtot + codes
        resid = d
        idx_cols.append(idx)
    q_ref[...] = qtot
    tok_ref[...] = jnp.concatenate(idx_cols, axis=1)
    loss_ref[...] = jnp.stack(loss_parts).reshape(1, 1, NQ)


@jax.jit
def kernel(z, codebooks):
    orig_shape = z.shape
    flat = z.reshape(-1, DIM)
    n = flat.shape[0]
    blk = 1024
    nblk = n // blk
    cn = jnp.sum(codebooks ** 2, axis=-1)

    q, tok, loss = pl.pallas_call(
        _rvq_kernel,
        grid=(nblk,),
        in_specs=[
            pl.BlockSpec((blk, DIM), lambda i: (i, 0)),
            pl.BlockSpec((NQ, CB, DIM), lambda i: (0, 0, 0)),
            pl.BlockSpec((NQ, CB), lambda i: (0, 0)),
        ],
        out_specs=[
            pl.BlockSpec((blk, DIM), lambda i: (i, 0)),
            pl.BlockSpec((blk, NQ), lambda i: (i, 0)),
            pl.BlockSpec((1, 1, NQ), lambda i: (i, 0, 0)),
        ],
        out_shape=[
            jax.ShapeDtypeStruct((n, DIM), jnp.float32),
            jax.ShapeDtypeStruct((n, NQ), jnp.int32),
            jax.ShapeDtypeStruct((nblk, 1, NQ), jnp.float32),
        ],
        compiler_params=pltpu.CompilerParams(
            dimension_semantics=("parallel",)),
    )(flat, codebooks, cn)

    quantized = q.reshape(orig_shape)
    tokens = tok.reshape(orig_shape[:-1] + (NQ,))
    commit_loss = jnp.sum(loss) * (1.25 / (NQ * n * DIM))
    return quantized, tokens, commit_loss


# speedup vs baseline: 1.8709x; 1.8709x over previous
import functools

import jax
import jax.numpy as jnp
from jax.experimental import pallas as pl
from jax.experimental.pallas import tpu as pltpu

DIM = 64
CB = 1024
NQ = 8


def _rvq_kernel(z_ref, cb_ref, cn_ref, cb1_ref, cb2_ref, cb3_ref,
                q_ref, tok_ref, loss_ref):
    resid = z_ref[...]
    blk = resid.shape[0]
    qtot = jnp.zeros_like(resid)
    iota = jax.lax.broadcasted_iota(jnp.int32, (blk, CB), 1)
    idx_cols = []
    loss_parts = []
    for s in range(NQ):
        cb = cb_ref[s]
        rn = jnp.sum(resid * resid, axis=1, keepdims=True)
        mm = jax.lax.dot_general(
            resid, cb, (((1,), (1,)), ((), ())),
            preferred_element_type=jnp.float32)
        dist = (rn - 2.0 * mm) + cn_ref[s][None, :]
        m = jnp.min(dist, axis=1, keepdims=True)
        idx = jnp.min(jnp.where(dist == m, iota, CB), axis=1, keepdims=True)
        oh = (iota == idx).astype(jnp.bfloat16)
        dn = (((1,), (0,)), ((), ()))
        c1 = jax.lax.dot_general(oh, cb1_ref[s], dn,
                                 preferred_element_type=jnp.float32)
        c2 = jax.lax.dot_general(oh, cb2_ref[s], dn,
                                 preferred_element_type=jnp.float32)
        c3 = jax.lax.dot_general(oh, cb3_ref[s], dn,
                                 preferred_element_type=jnp.float32)
        codes = (c1 + c2) + c3
        d = resid - codes
        loss_parts.append(jnp.sum(d * d))
        qtot = qtot + codes
        resid = d
        idx_cols.append(idx)
    q_ref[...] = qtot
    tok_ref[...] = jnp.concatenate(idx_cols, axis=1)
    loss_ref[...] = jnp.stack(loss_parts).reshape(1, 1, NQ)


@jax.jit
def kernel(z, codebooks):
    orig_shape = z.shape
    flat = z.reshape(-1, DIM)
    n = flat.shape[0]
    blk = 1024
    nblk = n // blk
    cn = jnp.sum(codebooks ** 2, axis=-1)
    cb1 = codebooks.astype(jnp.bfloat16)
    r1 = codebooks - cb1.astype(jnp.float32)
    cb2 = r1.astype(jnp.bfloat16)
    cb3 = (r1 - cb2.astype(jnp.float32)).astype(jnp.bfloat16)

    q, tok, loss = pl.pallas_call(
        _rvq_kernel,
        grid=(nblk,),
        in_specs=[
            pl.BlockSpec((blk, DIM), lambda i: (i, 0)),
            pl.BlockSpec((NQ, CB, DIM), lambda i: (0, 0, 0)),
            pl.BlockSpec((NQ, CB), lambda i: (0, 0)),
            pl.BlockSpec((NQ, CB, DIM), lambda i: (0, 0, 0)),
            pl.BlockSpec((NQ, CB, DIM), lambda i: (0, 0, 0)),
            pl.BlockSpec((NQ, CB, DIM), lambda i: (0, 0, 0)),
        ],
        out_specs=[
            pl.BlockSpec((blk, DIM), lambda i: (i, 0)),
            pl.BlockSpec((blk, NQ), lambda i: (i, 0)),
            pl.BlockSpec((1, 1, NQ), lambda i: (i, 0, 0)),
        ],
        out_shape=[
            jax.ShapeDtypeStruct((n, DIM), jnp.float32),
            jax.ShapeDtypeStruct((n, NQ), jnp.int32),
            jax.ShapeDtypeStruct((nblk, 1, NQ), jnp.float32),
        ],
        compiler_params=pltpu.CompilerParams(
            dimension_semantics=("parallel",)),
    )(flat, codebooks, cn, cb1, cb2, cb3)

    quantized = q.reshape(orig_shape)
    tokens = tok.reshape(orig_shape[:-1] + (NQ,))
    commit_loss = jnp.sum(loss) * (1.25 / (NQ * n * DIM))
    return quantized, tokens, commit_loss
